# baseline (speedup 1.0000x reference)
import functools

import jax
import jax.numpy as jnp
from jax import lax
from jax.experimental import pallas as pl
from jax.experimental.pallas import tpu as pltpu
from jax.experimental.pallas import tpu_sc as plsc

N_ITEMS = 100000
L = 8
TOTAL = N_ITEMS * L
N_CAT = 100000
D = 64
B = 16384

NW = 32
IPW = B // NW
CH = 16
NCH = IPW // CH
NBUF = 2
LANES = 16


def _make_kernel():
    mesh = plsc.VectorSubcoreMesh(core_axis_name="c", subcore_axis_name="s")

    @functools.partial(
        pl.kernel,
        out_type=jax.ShapeDtypeStruct((B, D), jnp.float32),
        mesh=mesh,
        compiler_params=pltpu.CompilerParams(needs_layout_passes=False,
                                             use_tc_tiling_on_sc=False),
        scratch_types=[
            pltpu.VMEM((4, 128), jnp.int32),
            pltpu.VMEM((4, 128), jnp.int32),
            pltpu.VMEM((IPW, 128), jnp.int32),
            pltpu.VMEM((NCH, 128), jnp.int32),
            pltpu.VMEM((NBUF, CH * L, D), jnp.float32),
            pltpu.VMEM((CH, D), jnp.float32),
            pltpu.SemaphoreType.DMA,
            pltpu.SemaphoreType.DMA((NBUF,)),
        ],
    )
    def embed_bag(items_hbm, ids_hbm, table_hbm, out_hbm,
                  items_v, items_q, ids_a, ids_v, rows_v, out_c, sem_i, sem_r):
        wid = lax.axis_index("s") * 2 + lax.axis_index("c")

        pltpu.sync_copy(items_hbm.at[wid], items_v)

        for g in range(4):
            for r in range(128 // LANES):
                sl = pl.ds(16 * r, 16)
                items_q[g, sl] = items_v[g, sl] >> 4

        for g in range(4):
            pltpu.async_copy(ids_hbm.at[items_q.at[g]],
                             ids_a.at[pl.ds(128 * g, 128)], sem_i)
        for g in range(4):
            pltpu.make_async_copy(ids_hbm.at[items_q.at[g]],
                                  ids_a.at[pl.ds(128 * g, 128)], sem_i).wait()

        lane = lax.iota(jnp.int32, LANES)

        def repack(c, carry):
            for r in range(128 // LANES):
                q = 128 * c + 16 * r + lane
                i = q >> 3
                v = plsc.load_gather(items_v, [i >> 7, i & 127])
                col = ((v & 15) << 3) | (q & 7)
                ids_v[c, pl.ds(16 * r, 16)] = plsc.load_gather(ids_a, [i, col])
            return carry

        lax.fori_loop(0, NCH, repack, 0)

        def start_rows(c, b):
            pltpu.async_copy(table_hbm.at[ids_v.at[c]], rows_v.at[b],
                             sem_r.at[b])

        def wait_rows(b):
            pltpu.make_async_copy(table_hbm.at[ids_v.at[0]],
                                  rows_v.at[b], sem_r.at[b]).wait()

        for b in range(NBUF):
            start_rows(b, b)

        def body(k, carry):
            for b in range(NBUF):
                c = k * NBUF + b
                wait_rows(b)
                for i in range(CH):
                    for g4 in range(D // LANES):
                        sl = pl.ds(LANES * g4, LANES)
                        acc = rows_v[b, L * i, sl]
                        for j in range(1, L):
                            acc = acc + rows_v[b, L * i + j, sl]
                        out_c[i, sl] = acc * 0.125
                pltpu.sync_copy(out_c,
                                out_hbm.at[pl.ds(wid * IPW + CH * c, CH)])
                nc = c + NBUF

                @pl.when(nc < NCH)
                def _():
                    start_rows(nc, b)
            return carry

        lax.fori_loop(0, NCH // NBUF, body, 0)

    return embed_bag


_embed_bag = _make_kernel()


def kernel(items, emb_bag_inputs, offsets, input_lengths, length_range,
           emb_weight):
    items_i = items.astype(jnp.int32).reshape(NW, 4, 128)
    ids_flat = emb_bag_inputs.astype(jnp.int32).reshape(TOTAL // 128, 128)
    return _embed_bag(items_i, ids_flat, emb_weight)


# speedup vs baseline: 1.4858x; 1.4858x over previous
import functools

import jax
import jax.numpy as jnp
from jax import lax
from jax.experimental import pallas as pl
from jax.experimental.pallas import tpu as pltpu
from jax.experimental.pallas import tpu_sc as plsc

N_ITEMS = 100000
L = 8
TOTAL = N_ITEMS * L
N_CAT = 100000
D = 64
B = 16384

NW = 32
IPW = B // NW
CH = 16
NCH = IPW // CH
NBUF = 2
LANES = 16


def _make_kernel():
    mesh = plsc.VectorSubcoreMesh(core_axis_name="c", subcore_axis_name="s")

    @functools.partial(
        pl.kernel,
        out_type=jax.ShapeDtypeStruct((B * D,), jnp.float32),
        mesh=mesh,
        compiler_params=pltpu.CompilerParams(needs_layout_passes=False,
                                             use_tc_tiling_on_sc=False),
        scratch_types=[
            pltpu.VMEM((4, 128), jnp.int32),
            pltpu.VMEM((4, 128), jnp.int32),
            pltpu.VMEM((IPW, 128), jnp.int32),
            pltpu.VMEM((NCH, 4, 128), jnp.int32),
            pltpu.VMEM((NBUF, 4, CH * L, LANES), jnp.float32),
            pltpu.VMEM((CH * D,), jnp.float32),
            pltpu.SemaphoreType.DMA,
            pltpu.SemaphoreType.DMA((NBUF,)),
        ],
    )
    def embed_bag(items_hbm, ids_hbm, table_hbm, out_hbm,
                  items_v, items_q, ids_a, ids_v, rows_v, out_c, sem_i, sem_r):
        wid = lax.axis_index("s") * 2 + lax.axis_index("c")

        pltpu.sync_copy(items_hbm.at[wid], items_v)

        for g in range(4):
            for r in range(128 // LANES):
                sl = pl.ds(16 * r, 16)
                items_q[g, sl] = items_v[g, sl] >> 4

        for g in range(4):
            pltpu.async_copy(ids_hbm.at[items_q.at[g]],
                             ids_a.at[pl.ds(128 * g, 128)], sem_i)
        for g in range(4):
            pltpu.make_async_copy(ids_hbm.at[items_q.at[g]],
                                  ids_a.at[pl.ds(128 * g, 128)], sem_i).wait()

        lane = lax.iota(jnp.int32, LANES)

        def repack(c, carry):
            for r in range(128 // LANES):
                q = 128 * c + 16 * r + lane
                i = q >> 3
                v = plsc.load_gather(items_v, [i >> 7, i & 127])
                col = ((v & 15) << 3) | (q & 7)
                v4 = plsc.load_gather(ids_a, [i, col]) << 2
                for g in range(4):
                    ids_v[c, g, pl.ds(16 * r, 16)] = v4 | g
            return carry

        lax.fori_loop(0, NCH, repack, 0)

        def start_rows(c, b):
            for g in range(4):
                pltpu.async_copy(table_hbm.at[ids_v.at[c, g]],
                                 rows_v.at[b, g], sem_r.at[b])

        def wait_rows(b):
            for g in range(4):
                pltpu.make_async_copy(table_hbm.at[ids_v.at[0, 0]],
                                      rows_v.at[b, g], sem_r.at[b]).wait()

        for b in range(NBUF):
            start_rows(b, b)

        def body(k, carry):
            for b in range(NBUF):
                c = k * NBUF + b
                wait_rows(b)
                for i in range(CH):
                    for g4 in range(D // LANES):
                        acc = rows_v[b, g4, L * i, :]
                        for j in range(1, L):
                            acc = acc + rows_v[b, g4, L * i + j, :]
                        out_c[pl.ds(D * i + LANES * g4, LANES)] = acc * 0.125
                pltpu.sync_copy(out_c,
                                out_hbm.at[pl.ds((wid * IPW + CH * c) * D,
                                                 CH * D)])
                nc = c + NBUF

                @pl.when(nc < NCH)
                def _():
                    start_rows(nc, b)
            return carry

        lax.fori_loop(0, NCH // NBUF, body, 0)

    return embed_bag


_embed_bag = _make_kernel()


def kernel(items, emb_bag_inputs, offsets, input_lengths, length_range,
           emb_weight):
    items_i = items.astype(jnp.int32).reshape(NW, 4, 128)
    ids_flat = emb_bag_inputs.astype(jnp.int32).reshape(TOTAL // 128, 128)
    table16 = emb_weight.reshape(N_CAT * D // 16, 16)
    return _embed_bag(items_i, ids_flat, table16).reshape(B, D)
